# baseline (speedup 1.0000x reference)
import functools

import jax
import jax.numpy as jnp
from jax import lax
from jax.experimental import pallas as pl
from jax.experimental.pallas import tpu as pltpu

_K = 3
_PAD = 1
_OG = 2
_K2 = _K * _K
_NT = _OG * _K2
_WIN = 4
_HB = 4


def _dc_kernel(xt_ref, xp_ref, wcat_ref, bcat_ref, wm_ref, out_ref,
               patch_ref, val_ref, idx_ref, wts_ref, *, H, W, C, Cg):
  h0 = pl.program_id(1) * _HB
  WB = _HB * W

  lane_c = lax.broadcasted_iota(jnp.int32, (C, W), 1)

  shifted = {}
  for dr in range(-_PAD, _HB + _PAD):
    row = h0 + dr
    rowc = jnp.clip(row, 0, H - 1)
    slab = xt_ref[0, rowc, :, :]
    valid = jnp.logical_and(row >= 0, row < H)
    slab = jnp.where(valid, slab, 0.0)
    sl = pltpu.roll(slab, 1, axis=1)
    sl = jnp.where(lane_c < 1, 0.0, sl)
    sr = pltpu.roll(slab, W - 1, axis=1)
    sr = jnp.where(lane_c >= W - 1, 0.0, sr)
    shifted[dr] = (sl, slab, sr)
  for ki in range(_K):
    for kj in range(_K):
      r0 = (ki * _K + kj) * C
      for hb in range(_HB):
        patch_ref[r0:r0 + C, hb * W:(hb + 1) * W] = shifted[hb + ki - _PAD][kj]

  om = jnp.dot(wcat_ref[...], patch_ref[...],
               preferred_element_type=jnp.float32) + bcat_ref[...]

  dy_all = om[0:_NT, :]
  dx_all = om[_NT:2 * _NT, :]
  m_all = jax.nn.sigmoid(om[2 * _NT:3 * _NT, :])

  si = lax.broadcasted_iota(jnp.int32, (_NT, WB), 0)
  lane_b = lax.broadcasted_iota(jnp.int32, (_NT, WB), 1)
  kiv = ((si % _K2) // _K).astype(jnp.float32)
  kjv = (si % _K).astype(jnp.float32)
  hbv = (lane_b // W).astype(jnp.float32)
  wv = (lane_b % W).astype(jnp.float32)

  h0f = h0.astype(jnp.float32)
  py = dy_all + (h0f - _PAD) + hbv + kiv
  px = dx_all + (wv - _PAD) + kjv
  y0f = jnp.floor(py)
  x0f = jnp.floor(px)
  wx = px - x0f
  x0 = x0f.astype(jnp.int32)
  x1 = x0 + 1
  x0c = jnp.clip(x0, 0, W - 1)
  x1c = jnp.clip(x1, 0, W - 1)
  vx0 = jnp.where(jnp.logical_and(x0 >= 0, x0 <= W - 1), 1.0, 0.0)
  vx1 = jnp.where(jnp.logical_and(x1 >= 0, x1 <= W - 1), 1.0, 0.0)
  mwxl = (1.0 - wx) * vx0 * m_all
  mwxr = wx * vx1 * m_all
  a0 = mwxl + jnp.where(x0 == -1, mwxr, 0.0)
  a1 = jnp.where(x0 == -1, 0.0, mwxr)

  idx_ref[...] = x0c
  wts_ref[0:_NT, :] = a0
  wts_ref[_NT:2 * _NT, :] = a1
  wts_ref[2 * _NT:3 * _NT, :] = py

  base_f = []
  hi_f = []
  span = jnp.float32(0.0)
  for hb in range(_HB):
    ys = y0f[:, hb * W:(hb + 1) * W]
    ymin = jnp.min(ys, axis=1, keepdims=True)
    ymax = jnp.max(ys, axis=1, keepdims=True)
    b_ = jnp.clip(ymin, 0.0, float(H - _WIN))
    l_ = jnp.clip(ymin, 0.0, float(H - 1))
    hi_ = jnp.clip(ymax + 1.0, 0.0, float(H - 1))
    base_f.append(b_)
    hi_f.append(hi_)
    span = jnp.maximum(span, jnp.max(hi_ - l_))

  def contrib(rf, slab, i, hb, x0cb, pyr, a0r, a1r):
    tp = jnp.take_along_axis(slab, x0cb, axis=1)
    t0 = pltpu.bitcast(tp << 16, jnp.float32)
    t1 = pltpu.bitcast(tp & jnp.int32(-65536), jnp.float32)
    cy = jnp.maximum(1.0 - jnp.abs(pyr - rf), 0.0)
    la = jnp.broadcast_to(a0r * cy, (Cg, W))
    ra = jnp.broadcast_to(a1r * cy, (Cg, W))
    return t0 * la + t1 * ra

  for g in range(_OG):
    gs = g * Cg
    for k in range(_K2):
      i = g * _K2 + k
      for hb in range(_HB):
        cs = slice(hb * W, (hb + 1) * W)
        x0cb = jnp.broadcast_to(idx_ref[i:i + 1, cs], (Cg, W))
        pyr = wts_ref[2 * _NT + i:2 * _NT + i + 1, cs]
        a0r = wts_ref[i:i + 1, cs]
        a1r = wts_ref[_NT + i:_NT + i + 1, cs]
        base = base_f[hb][i, 0].astype(jnp.int32)
        slab4 = xp_ref[0, pl.ds(base, _WIN), gs:gs + Cg, :]

        acc = jnp.zeros((Cg, W), jnp.float32)
        for u in range(_WIN):
          rf = (base + u).astype(jnp.float32)
          acc = acc + contrib(rf, slab4[u], i, hb, x0cb, pyr, a0r, a1r)

        val_ref[i * Cg:(i + 1) * Cg, cs] = acc

  @pl.when(span > float(_WIN) - 0.5)
  def _residual():
    for g in range(_OG):
      gs = g * Cg
      for k in range(_K2):
        i = g * _K2 + k
        for hb in range(_HB):
          cs = slice(hb * W, (hb + 1) * W)
          x0cb = jnp.broadcast_to(idx_ref[i:i + 1, cs], (Cg, W))
          pyr = wts_ref[2 * _NT + i:2 * _NT + i + 1, cs]
          a0r = wts_ref[i:i + 1, cs]
          a1r = wts_ref[_NT + i:_NT + i + 1, cs]
          base = base_f[hb][i, 0].astype(jnp.int32)
          hi = hi_f[hb][i, 0].astype(jnp.int32)

          def body(r, acc, *, gs=gs, x0cb=x0cb, pyr=pyr, a0r=a0r, a1r=a1r):
            slab = xp_ref[0, r, gs:gs + Cg, :]
            return acc + contrib(r.astype(jnp.float32), slab, 0, 0, x0cb,
                                 pyr, a0r, a1r)

          acc = lax.fori_loop(base + _WIN, hi + 1, body,
                              jnp.zeros((Cg, W), jnp.float32))
          val_ref[i * Cg:(i + 1) * Cg, cs] = (
              val_ref[i * Cg:(i + 1) * Cg, cs] + acc)

  res = jnp.dot(wm_ref[...], val_ref[...],
                preferred_element_type=jnp.float32)
  for hb in range(_HB):
    out_ref[0, hb, :, :] = res[:, hb * W:(hb + 1) * W]


@jax.jit
def kernel(x, w_main, w_off, b_off, w_mask, b_mask):
  B, C, H, W = x.shape
  O = w_main.shape[0]
  Cg = C // _OG
  n_cat = 3 * _NT
  n_pad = 56

  xt = jnp.transpose(x, (0, 2, 1, 3))

  xtb = xt.astype(jnp.bfloat16)
  xlo = lax.bitcast_convert_type(xtb, jnp.uint16).astype(jnp.uint32)
  xnb = jnp.pad(xtb[:, :, :, 1:], ((0, 0), (0, 0), (0, 0), (0, 1)))
  xhi = lax.bitcast_convert_type(xnb, jnp.uint16).astype(jnp.uint32)
  xp = lax.bitcast_convert_type((xhi << 16) | xlo, jnp.int32)

  w_off_r = w_off.reshape(_NT, 2, C, _K, _K)
  b_off_r = b_off.reshape(_NT, 2)
  wcat = jnp.concatenate([w_off_r[:, 0], w_off_r[:, 1], w_mask], axis=0)
  wcat = wcat.transpose(0, 2, 3, 1).reshape(n_cat, _K2 * C)
  wcat = jnp.pad(wcat, ((0, n_pad - n_cat), (0, 0)))
  bcat = jnp.concatenate([b_off_r[:, 0], b_off_r[:, 1], b_mask], axis=0)
  bcat = jnp.pad(bcat, (0, n_pad - n_cat))
  bcat = jnp.broadcast_to(bcat[:, None], (n_pad, _HB * W))

  wm = w_main.reshape(O, _OG, Cg, _K, _K)
  wm = wm.transpose(0, 1, 3, 4, 2).reshape(O, _NT * Cg)

  body = functools.partial(_dc_kernel, H=H, W=W, C=C, Cg=Cg)
  out_t = pl.pallas_call(
      body,
      grid=(B, H // _HB),
      in_specs=[
          pl.BlockSpec((1, H, C, W), lambda b, j: (b, 0, 0, 0)),
          pl.BlockSpec((1, H, C, W), lambda b, j: (b, 0, 0, 0)),
          pl.BlockSpec((n_pad, _K2 * C), lambda b, j: (0, 0)),
          pl.BlockSpec((n_pad, _HB * W), lambda b, j: (0, 0)),
          pl.BlockSpec((O, _NT * Cg), lambda b, j: (0, 0)),
      ],
      out_specs=pl.BlockSpec((1, _HB, O, W), lambda b, j: (b, j, 0, 0)),
      out_shape=jax.ShapeDtypeStruct((B, H, O, W), jnp.float32),
      scratch_shapes=[
          pltpu.VMEM((_K2 * C, _HB * W), jnp.float32),
          pltpu.VMEM((_NT * Cg, _HB * W), jnp.float32),
          pltpu.VMEM((_NT, _HB * W), jnp.int32),
          pltpu.VMEM((3 * _NT, _HB * W), jnp.float32),
      ],
      compiler_params=pltpu.CompilerParams(
          dimension_semantics=(pltpu.GridDimensionSemantics.PARALLEL,
                               pltpu.GridDimensionSemantics.ARBITRARY),
          vmem_limit_bytes=64 * 1024 * 1024,
      ),
  )(xt, xp, wcat, bcat, wm)

  return jnp.transpose(out_t, (0, 2, 1, 3))


# speedup vs baseline: 1.1565x; 1.1565x over previous
import functools

import jax
import jax.numpy as jnp
from jax import lax
from jax.experimental import pallas as pl
from jax.experimental.pallas import tpu as pltpu

_K = 3
_PAD = 1
_OG = 2
_K2 = _K * _K
_NT = _OG * _K2
_WIN = 4
_HB = 8


def _dc_kernel(xt_ref, xp_ref, wcat_ref, bcat_ref, wm_ref, out_ref,
               patch_ref, val_ref, idx_ref, wts_ref, *, H, W, C, Cg):
  h0 = pl.program_id(1) * _HB
  WB = _HB * W

  lane_c = lax.broadcasted_iota(jnp.int32, (C, W), 1)

  shifted = {}
  for dr in range(-_PAD, _HB + _PAD):
    row = h0 + dr
    rowc = jnp.clip(row, 0, H - 1)
    slab = xt_ref[0, rowc, :, :]
    valid = jnp.logical_and(row >= 0, row < H)
    slab = jnp.where(valid, slab, 0.0)
    sl = pltpu.roll(slab, 1, axis=1)
    sl = jnp.where(lane_c < 1, 0.0, sl)
    sr = pltpu.roll(slab, W - 1, axis=1)
    sr = jnp.where(lane_c >= W - 1, 0.0, sr)
    shifted[dr] = (sl, slab, sr)
  for ki in range(_K):
    for kj in range(_K):
      r0 = (ki * _K + kj) * C
      for hb in range(_HB):
        patch_ref[r0:r0 + C, hb * W:(hb + 1) * W] = shifted[hb + ki - _PAD][kj]

  om = jnp.dot(wcat_ref[...], patch_ref[...],
               preferred_element_type=jnp.float32) + bcat_ref[...]

  dy_all = om[0:_NT, :]
  dx_all = om[_NT:2 * _NT, :]
  m_all = jax.nn.sigmoid(om[2 * _NT:3 * _NT, :])

  si = lax.broadcasted_iota(jnp.int32, (_NT, WB), 0)
  lane_b = lax.broadcasted_iota(jnp.int32, (_NT, WB), 1)
  kiv = ((si % _K2) // _K).astype(jnp.float32)
  kjv = (si % _K).astype(jnp.float32)
  hbv = (lane_b // W).astype(jnp.float32)
  wv = (lane_b % W).astype(jnp.float32)

  h0f = h0.astype(jnp.float32)
  py = dy_all + (h0f - _PAD) + hbv + kiv
  px = dx_all + (wv - _PAD) + kjv
  y0f = jnp.floor(py)
  x0f = jnp.floor(px)
  wx = px - x0f
  x0 = x0f.astype(jnp.int32)
  x1 = x0 + 1
  x0c = jnp.clip(x0, 0, W - 1)
  x1c = jnp.clip(x1, 0, W - 1)
  vx0 = jnp.where(jnp.logical_and(x0 >= 0, x0 <= W - 1), 1.0, 0.0)
  vx1 = jnp.where(jnp.logical_and(x1 >= 0, x1 <= W - 1), 1.0, 0.0)
  mwxl = (1.0 - wx) * vx0 * m_all
  mwxr = wx * vx1 * m_all
  a0 = mwxl + jnp.where(x0 == -1, mwxr, 0.0)
  a1 = jnp.where(x0 == -1, 0.0, mwxr)

  idx_ref[...] = x0c
  wts_ref[0:_NT, :] = a0
  wts_ref[_NT:2 * _NT, :] = a1
  wts_ref[2 * _NT:3 * _NT, :] = py

  base_f = []
  hi_f = []
  span = jnp.float32(0.0)
  for hb in range(_HB):
    ys = y0f[:, hb * W:(hb + 1) * W]
    ymin = jnp.min(ys, axis=1, keepdims=True)
    ymax = jnp.max(ys, axis=1, keepdims=True)
    b_ = jnp.clip(ymin, 0.0, float(H - _WIN))
    l_ = jnp.clip(ymin, 0.0, float(H - 1))
    hi_ = jnp.clip(ymax + 1.0, 0.0, float(H - 1))
    base_f.append(b_)
    hi_f.append(hi_)
    span = jnp.maximum(span, jnp.max(hi_ - l_))

  def contrib(rf, slab, i, hb, x0cb, pyr, a0r, a1r):
    tp = jnp.take_along_axis(slab, x0cb, axis=1)
    t0 = pltpu.bitcast(tp << 16, jnp.float32)
    t1 = pltpu.bitcast(tp & jnp.int32(-65536), jnp.float32)
    cy = jnp.maximum(1.0 - jnp.abs(pyr - rf), 0.0)
    la = jnp.broadcast_to(a0r * cy, (Cg, W))
    ra = jnp.broadcast_to(a1r * cy, (Cg, W))
    return t0 * la + t1 * ra

  for g in range(_OG):
    gs = g * Cg
    for k in range(_K2):
      i = g * _K2 + k
      for hb in range(_HB):
        cs = slice(hb * W, (hb + 1) * W)
        x0cb = jnp.broadcast_to(idx_ref[i:i + 1, cs], (Cg, W))
        pyr = wts_ref[2 * _NT + i:2 * _NT + i + 1, cs]
        a0r = wts_ref[i:i + 1, cs]
        a1r = wts_ref[_NT + i:_NT + i + 1, cs]
        base = base_f[hb][i, 0].astype(jnp.int32)
        slab4 = xp_ref[0, pl.ds(base, _WIN), gs:gs + Cg, :]

        acc = jnp.zeros((Cg, W), jnp.float32)
        for u in range(_WIN):
          rf = (base + u).astype(jnp.float32)
          acc = acc + contrib(rf, slab4[u], i, hb, x0cb, pyr, a0r, a1r)

        val_ref[i * Cg:(i + 1) * Cg, cs] = acc

  @pl.when(span > float(_WIN) - 0.5)
  def _residual():
    for g in range(_OG):
      gs = g * Cg
      for k in range(_K2):
        i = g * _K2 + k
        for hb in range(_HB):
          cs = slice(hb * W, (hb + 1) * W)
          x0cb = jnp.broadcast_to(idx_ref[i:i + 1, cs], (Cg, W))
          pyr = wts_ref[2 * _NT + i:2 * _NT + i + 1, cs]
          a0r = wts_ref[i:i + 1, cs]
          a1r = wts_ref[_NT + i:_NT + i + 1, cs]
          base = base_f[hb][i, 0].astype(jnp.int32)
          hi = hi_f[hb][i, 0].astype(jnp.int32)

          def body(r, acc, *, gs=gs, x0cb=x0cb, pyr=pyr, a0r=a0r, a1r=a1r):
            slab = xp_ref[0, r, gs:gs + Cg, :]
            return acc + contrib(r.astype(jnp.float32), slab, 0, 0, x0cb,
                                 pyr, a0r, a1r)

          acc = lax.fori_loop(base + _WIN, hi + 1, body,
                              jnp.zeros((Cg, W), jnp.float32))
          val_ref[i * Cg:(i + 1) * Cg, cs] = (
              val_ref[i * Cg:(i + 1) * Cg, cs] + acc)

  res = jnp.dot(wm_ref[...], val_ref[...],
                preferred_element_type=jnp.float32)
  for hb in range(_HB):
    out_ref[0, :, hb, :] = res[:, hb * W:(hb + 1) * W]


@jax.jit
def kernel(x, w_main, w_off, b_off, w_mask, b_mask):
  B, C, H, W = x.shape
  O = w_main.shape[0]
  Cg = C // _OG
  n_cat = 3 * _NT
  n_pad = 56

  xt = jnp.transpose(x, (0, 2, 1, 3))

  xtb = xt.astype(jnp.bfloat16)
  xlo = lax.bitcast_convert_type(xtb, jnp.uint16).astype(jnp.uint32)
  xnb = jnp.pad(xtb[:, :, :, 1:], ((0, 0), (0, 0), (0, 0), (0, 1)))
  xhi = lax.bitcast_convert_type(xnb, jnp.uint16).astype(jnp.uint32)
  xp = lax.bitcast_convert_type((xhi << 16) | xlo, jnp.int32)

  w_off_r = w_off.reshape(_NT, 2, C, _K, _K)
  b_off_r = b_off.reshape(_NT, 2)
  wcat = jnp.concatenate([w_off_r[:, 0], w_off_r[:, 1], w_mask], axis=0)
  wcat = wcat.transpose(0, 2, 3, 1).reshape(n_cat, _K2 * C)
  wcat = jnp.pad(wcat, ((0, n_pad - n_cat), (0, 0)))
  bcat = jnp.concatenate([b_off_r[:, 0], b_off_r[:, 1], b_mask], axis=0)
  bcat = jnp.pad(bcat, (0, n_pad - n_cat))
  bcat = jnp.broadcast_to(bcat[:, None], (n_pad, _HB * W))

  wm = w_main.reshape(O, _OG, Cg, _K, _K)
  wm = wm.transpose(0, 1, 3, 4, 2).reshape(O, _NT * Cg)

  body = functools.partial(_dc_kernel, H=H, W=W, C=C, Cg=Cg)
  out_t = pl.pallas_call(
      body,
      grid=(B, H // _HB),
      in_specs=[
          pl.BlockSpec((1, H, C, W), lambda b, j: (b, 0, 0, 0)),
          pl.BlockSpec((1, H, C, W), lambda b, j: (b, 0, 0, 0)),
          pl.BlockSpec((n_pad, _K2 * C), lambda b, j: (0, 0)),
          pl.BlockSpec((n_pad, _HB * W), lambda b, j: (0, 0)),
          pl.BlockSpec((O, _NT * Cg), lambda b, j: (0, 0)),
      ],
      out_specs=pl.BlockSpec((1, O, _HB, W), lambda b, j: (b, 0, j, 0)),
      out_shape=jax.ShapeDtypeStruct((B, O, H, W), jnp.float32),
      scratch_shapes=[
          pltpu.VMEM((_K2 * C, _HB * W), jnp.float32),
          pltpu.VMEM((_NT * Cg, _HB * W), jnp.float32),
          pltpu.VMEM((_NT, _HB * W), jnp.int32),
          pltpu.VMEM((3 * _NT, _HB * W), jnp.float32),
      ],
      compiler_params=pltpu.CompilerParams(
          dimension_semantics=(pltpu.GridDimensionSemantics.PARALLEL,
                               pltpu.GridDimensionSemantics.ARBITRARY),
          vmem_limit_bytes=64 * 1024 * 1024,
      ),
  )(xt, xp, wcat, bcat, wm)

  return out_t


# speedup vs baseline: 1.2099x; 1.0461x over previous
import functools

import jax
import jax.numpy as jnp
from jax import lax
from jax.experimental import pallas as pl
from jax.experimental.pallas import tpu as pltpu

_K = 3
_PAD = 1
_OG = 2
_K2 = _K * _K
_NT = _OG * _K2
_WIN = 4
_HB = 8


def _dc_kernel(xp_ref, wcat_ref, bcat_ref, wm_ref, out_ref,
               patch_ref, val_ref, idx_ref, wts_ref, *, H, W, C, Cg):
  h0 = pl.program_id(1) * _HB
  WB = _HB * W

  lane_c = lax.broadcasted_iota(jnp.int32, (C, W), 1)

  shifted = {}
  for dr in range(-_PAD, _HB + _PAD):
    row = h0 + dr
    rowc = jnp.clip(row, 0, H - 1)
    slab_i = xp_ref[0, rowc, :, :]
    valid = jnp.logical_and(row >= 0, row < H)
    slab_i = jnp.where(valid, slab_i, 0)
    slab = pltpu.bitcast(slab_i << 16, jnp.float32)
    sr = pltpu.bitcast(slab_i & jnp.int32(-65536), jnp.float32)
    sl = pltpu.roll(slab, 1, axis=1)
    sl = jnp.where(lane_c < 1, 0.0, sl)
    shifted[dr] = (sl, slab, sr)
  for ki in range(_K):
    for kj in range(_K):
      r0 = (ki * _K + kj) * C
      for hb in range(_HB):
        patch_ref[r0:r0 + C, hb * W:(hb + 1) * W] = shifted[hb + ki - _PAD][kj]

  om = jnp.dot(wcat_ref[...], patch_ref[...],
               preferred_element_type=jnp.float32) + bcat_ref[...]

  dy_all = om[0:_NT, :]
  dx_all = om[_NT:2 * _NT, :]
  m_all = jax.nn.sigmoid(om[2 * _NT:3 * _NT, :])

  si = lax.broadcasted_iota(jnp.int32, (_NT, WB), 0)
  lane_b = lax.broadcasted_iota(jnp.int32, (_NT, WB), 1)
  kiv = ((si % _K2) // _K).astype(jnp.float32)
  kjv = (si % _K).astype(jnp.float32)
  hbv = (lane_b // W).astype(jnp.float32)
  wv = (lane_b % W).astype(jnp.float32)

  h0f = h0.astype(jnp.float32)
  py = dy_all + (h0f - _PAD) + hbv + kiv
  px = dx_all + (wv - _PAD) + kjv
  y0f = jnp.floor(py)
  x0f = jnp.floor(px)
  wx = px - x0f
  x0 = x0f.astype(jnp.int32)
  x1 = x0 + 1
  x0c = jnp.clip(x0, 0, W - 1)
  x1c = jnp.clip(x1, 0, W - 1)
  vx0 = jnp.where(jnp.logical_and(x0 >= 0, x0 <= W - 1), 1.0, 0.0)
  vx1 = jnp.where(jnp.logical_and(x1 >= 0, x1 <= W - 1), 1.0, 0.0)
  mwxl = (1.0 - wx) * vx0 * m_all
  mwxr = wx * vx1 * m_all
  a0 = mwxl + jnp.where(x0 == -1, mwxr, 0.0)
  a1 = jnp.where(x0 == -1, 0.0, mwxr)

  idx_ref[...] = x0c
  wts_ref[0:_NT, :] = a0
  wts_ref[_NT:2 * _NT, :] = a1
  wts_ref[2 * _NT:3 * _NT, :] = py

  base_f = []
  hi_f = []
  span = jnp.float32(0.0)
  for hb in range(_HB):
    ys = y0f[:, hb * W:(hb + 1) * W]
    ymin = jnp.min(ys, axis=1, keepdims=True)
    ymax = jnp.max(ys, axis=1, keepdims=True)
    b_ = jnp.clip(ymin, 0.0, float(H - _WIN))
    l_ = jnp.clip(ymin, 0.0, float(H - 1))
    hi_ = jnp.clip(ymax + 1.0, 0.0, float(H - 1))
    base_f.append(b_)
    hi_f.append(hi_)
    span = jnp.maximum(span, jnp.max(hi_ - l_))

  def contrib(rf, slab, i, hb, x0cb, pyr, a0r, a1r):
    tp = jnp.take_along_axis(slab, x0cb, axis=1)
    t0 = pltpu.bitcast(tp << 16, jnp.float32)
    t1 = pltpu.bitcast(tp & jnp.int32(-65536), jnp.float32)
    cy = jnp.maximum(1.0 - jnp.abs(pyr - rf), 0.0)
    la = jnp.broadcast_to(a0r * cy, (Cg, W))
    ra = jnp.broadcast_to(a1r * cy, (Cg, W))
    return t0 * la + t1 * ra

  for g in range(_OG):
    gs = g * Cg
    for k in range(_K2):
      i = g * _K2 + k
      for hb in range(_HB):
        cs = slice(hb * W, (hb + 1) * W)
        x0cb = jnp.broadcast_to(idx_ref[i:i + 1, cs], (Cg, W))
        pyr = wts_ref[2 * _NT + i:2 * _NT + i + 1, cs]
        a0r = wts_ref[i:i + 1, cs]
        a1r = wts_ref[_NT + i:_NT + i + 1, cs]
        base = base_f[hb][i, 0].astype(jnp.int32)
        slab4 = xp_ref[0, pl.ds(base, _WIN), gs:gs + Cg, :]

        acc = jnp.zeros((Cg, W), jnp.float32)
        for u in range(_WIN):
          rf = (base + u).astype(jnp.float32)
          acc = acc + contrib(rf, slab4[u], i, hb, x0cb, pyr, a0r, a1r)

        val_ref[i * Cg:(i + 1) * Cg, cs] = acc

  @pl.when(span > float(_WIN) - 0.5)
  def _residual():
    for g in range(_OG):
      gs = g * Cg
      for k in range(_K2):
        i = g * _K2 + k
        for hb in range(_HB):
          cs = slice(hb * W, (hb + 1) * W)
          x0cb = jnp.broadcast_to(idx_ref[i:i + 1, cs], (Cg, W))
          pyr = wts_ref[2 * _NT + i:2 * _NT + i + 1, cs]
          a0r = wts_ref[i:i + 1, cs]
          a1r = wts_ref[_NT + i:_NT + i + 1, cs]
          base = base_f[hb][i, 0].astype(jnp.int32)
          hi = hi_f[hb][i, 0].astype(jnp.int32)

          def body(r, acc, *, gs=gs, x0cb=x0cb, pyr=pyr, a0r=a0r, a1r=a1r):
            slab = xp_ref[0, r, gs:gs + Cg, :]
            return acc + contrib(r.astype(jnp.float32), slab, 0, 0, x0cb,
                                 pyr, a0r, a1r)

          acc = lax.fori_loop(base + _WIN, hi + 1, body,
                              jnp.zeros((Cg, W), jnp.float32))
          val_ref[i * Cg:(i + 1) * Cg, cs] = (
              val_ref[i * Cg:(i + 1) * Cg, cs] + acc)

  res = jnp.dot(wm_ref[...], val_ref[...],
                preferred_element_type=jnp.float32)
  for hb in range(_HB):
    out_ref[0, :, hb, :] = res[:, hb * W:(hb + 1) * W]


@jax.jit
def kernel(x, w_main, w_off, b_off, w_mask, b_mask):
  B, C, H, W = x.shape
  O = w_main.shape[0]
  Cg = C // _OG
  n_cat = 3 * _NT
  n_pad = 56

  xb = x.astype(jnp.bfloat16)
  xlo = lax.bitcast_convert_type(xb, jnp.uint16).astype(jnp.uint32)
  xnb = jnp.pad(xb[:, :, :, 1:], ((0, 0), (0, 0), (0, 0), (0, 1)))
  xhi = lax.bitcast_convert_type(xnb, jnp.uint16).astype(jnp.uint32)
  xpk = lax.bitcast_convert_type((xhi << 16) | xlo, jnp.int32)
  xp = jnp.transpose(xpk, (0, 2, 1, 3))

  w_off_r = w_off.reshape(_NT, 2, C, _K, _K)
  b_off_r = b_off.reshape(_NT, 2)
  wcat = jnp.concatenate([w_off_r[:, 0], w_off_r[:, 1], w_mask], axis=0)
  wcat = wcat.transpose(0, 2, 3, 1).reshape(n_cat, _K2 * C)
  wcat = jnp.pad(wcat, ((0, n_pad - n_cat), (0, 0)))
  bcat = jnp.concatenate([b_off_r[:, 0], b_off_r[:, 1], b_mask], axis=0)
  bcat = jnp.pad(bcat, (0, n_pad - n_cat))
  bcat = jnp.broadcast_to(bcat[:, None], (n_pad, _HB * W))

  wm = w_main.reshape(O, _OG, Cg, _K, _K)
  wm = wm.transpose(0, 1, 3, 4, 2).reshape(O, _NT * Cg)

  body = functools.partial(_dc_kernel, H=H, W=W, C=C, Cg=Cg)
  out_t = pl.pallas_call(
      body,
      grid=(B, H // _HB),
      in_specs=[
          pl.BlockSpec((1, H, C, W), lambda b, j: (b, 0, 0, 0)),
          pl.BlockSpec((n_pad, _K2 * C), lambda b, j: (0, 0)),
          pl.BlockSpec((n_pad, _HB * W), lambda b, j: (0, 0)),
          pl.BlockSpec((O, _NT * Cg), lambda b, j: (0, 0)),
      ],
      out_specs=pl.BlockSpec((1, O, _HB, W), lambda b, j: (b, 0, j, 0)),
      out_shape=jax.ShapeDtypeStruct((B, O, H, W), jnp.float32),
      scratch_shapes=[
          pltpu.VMEM((_K2 * C, _HB * W), jnp.float32),
          pltpu.VMEM((_NT * Cg, _HB * W), jnp.float32),
          pltpu.VMEM((_NT, _HB * W), jnp.int32),
          pltpu.VMEM((3 * _NT, _HB * W), jnp.float32),
      ],
      compiler_params=pltpu.CompilerParams(
          dimension_semantics=(pltpu.GridDimensionSemantics.PARALLEL,
                               pltpu.GridDimensionSemantics.ARBITRARY),
          vmem_limit_bytes=64 * 1024 * 1024,
      ),
  )(xp, wcat, bcat, wm)

  return out_t


# speedup vs baseline: 1.2138x; 1.0032x over previous
import functools

import jax
import jax.numpy as jnp
from jax import lax
from jax.experimental import pallas as pl
from jax.experimental.pallas import tpu as pltpu

_K = 3
_PAD = 1
_OG = 2
_K2 = _K * _K
_NT = _OG * _K2
_WIN = 4
_HB = 8


def _dc_kernel(xp_ref, wcat_ref, bcat_ref, wm_ref, out_ref,
               patch_ref, val_ref, idx_ref, wts_ref, *, H, W, C, Cg):
  h0 = pl.program_id(1) * _HB
  WB = _HB * W

  lane_c = lax.broadcasted_iota(jnp.int32, (C, W), 1)

  shifted = {}
  for dr in range(-_PAD, _HB + _PAD):
    row = h0 + dr
    rowc = jnp.clip(row, 0, H - 1)
    slab_i = xp_ref[0, rowc, :, :]
    valid = jnp.logical_and(row >= 0, row < H)
    slab_i = jnp.where(valid, slab_i, 0)
    slab = pltpu.bitcast(slab_i << 16, jnp.float32)
    sr = pltpu.bitcast(slab_i & jnp.int32(-65536), jnp.float32)
    sl = pltpu.roll(slab, 1, axis=1)
    sl = jnp.where(lane_c < 1, 0.0, sl)
    shifted[dr] = (sl, slab, sr)
  for ki in range(_K):
    for kj in range(_K):
      r0 = (ki * _K + kj) * C
      for hb in range(_HB):
        patch_ref[r0:r0 + C, hb * W:(hb + 1) * W] = shifted[hb + ki - _PAD][kj]

  om = jnp.dot(wcat_ref[...], patch_ref[...],
               preferred_element_type=jnp.float32) + bcat_ref[...]

  dy_all = om[0:_NT, :]
  dx_all = om[_NT:2 * _NT, :]
  m_all = jax.nn.sigmoid(om[2 * _NT:3 * _NT, :])

  si = lax.broadcasted_iota(jnp.int32, (_NT, WB), 0)
  lane_b = lax.broadcasted_iota(jnp.int32, (_NT, WB), 1)
  kiv = ((si % _K2) // _K).astype(jnp.float32)
  kjv = (si % _K).astype(jnp.float32)
  hbv = (lane_b // W).astype(jnp.float32)
  wv = (lane_b % W).astype(jnp.float32)

  h0f = h0.astype(jnp.float32)
  py = dy_all + (h0f - _PAD) + hbv + kiv
  px = dx_all + (wv - _PAD) + kjv
  y0f = jnp.floor(py)
  x0f = jnp.floor(px)
  wx = px - x0f
  x0 = x0f.astype(jnp.int32)
  x1 = x0 + 1
  x0c = jnp.clip(x0, 0, W - 1)
  x1c = jnp.clip(x1, 0, W - 1)
  vx0 = jnp.where(jnp.logical_and(x0 >= 0, x0 <= W - 1), 1.0, 0.0)
  vx1 = jnp.where(jnp.logical_and(x1 >= 0, x1 <= W - 1), 1.0, 0.0)
  mwxl = (1.0 - wx) * vx0 * m_all
  mwxr = wx * vx1 * m_all
  a0 = mwxl + jnp.where(x0 == -1, mwxr, 0.0)
  a1 = jnp.where(x0 == -1, 0.0, mwxr)

  idx_ref[...] = x0c
  wts_ref[0:_NT, :] = a0
  wts_ref[_NT:2 * _NT, :] = a1
  wts_ref[2 * _NT:3 * _NT, :] = py

  base_f = []
  hi_f = []
  span = jnp.float32(0.0)
  for hb in range(_HB):
    ys = y0f[:, hb * W:(hb + 1) * W]
    ymin = jnp.min(ys, axis=1, keepdims=True)
    ymax = jnp.max(ys, axis=1, keepdims=True)
    b_ = jnp.clip(ymin, 0.0, float(H - _WIN))
    l_ = jnp.clip(ymin, 0.0, float(H - 1))
    hi_ = jnp.clip(ymax + 1.0, 0.0, float(H - 1))
    base_f.append(b_)
    hi_f.append(hi_)
    span = jnp.maximum(span, jnp.max(hi_ - l_))

  def contrib(rf, slab, i, hb, x0cb, pyr, a0r, a1r):
    tp = jnp.take_along_axis(slab, x0cb, axis=1)
    t0 = pltpu.bitcast(tp << 16, jnp.float32)
    t1 = pltpu.bitcast(tp & jnp.int32(-65536), jnp.float32)
    cy = jnp.maximum(1.0 - jnp.abs(pyr - rf), 0.0)
    la = jnp.broadcast_to(a0r * cy, (Cg, W))
    ra = jnp.broadcast_to(a1r * cy, (Cg, W))
    return t0 * la + t1 * ra

  for g in range(_OG):
    gs = g * Cg
    for k in range(_K2):
      i = g * _K2 + k
      for hb in range(_HB):
        cs = slice(hb * W, (hb + 1) * W)
        x0cb = jnp.broadcast_to(idx_ref[i:i + 1, cs], (Cg, W))
        pyr = wts_ref[2 * _NT + i:2 * _NT + i + 1, cs]
        a0r = wts_ref[i:i + 1, cs]
        a1r = wts_ref[_NT + i:_NT + i + 1, cs]
        base = base_f[hb][i, 0].astype(jnp.int32)
        slab4 = xp_ref[0, pl.ds(base, _WIN), gs:gs + Cg, :]

        acc = jnp.zeros((Cg, W), jnp.float32)
        for u in range(_WIN):
          rf = (base + u).astype(jnp.float32)
          acc = acc + contrib(rf, slab4[u], i, hb, x0cb, pyr, a0r, a1r)

        val_ref[i * Cg:(i + 1) * Cg, cs] = acc

  @pl.when(span > float(_WIN) - 0.5)
  def _residual():
    for g in range(_OG):
      gs = g * Cg
      for k in range(_K2):
        i = g * _K2 + k
        for hb in range(_HB):
          cs = slice(hb * W, (hb + 1) * W)
          x0cb = jnp.broadcast_to(idx_ref[i:i + 1, cs], (Cg, W))
          pyr = wts_ref[2 * _NT + i:2 * _NT + i + 1, cs]
          a0r = wts_ref[i:i + 1, cs]
          a1r = wts_ref[_NT + i:_NT + i + 1, cs]
          base = base_f[hb][i, 0].astype(jnp.int32)
          hi = hi_f[hb][i, 0].astype(jnp.int32)

          def body(r, acc, *, gs=gs, x0cb=x0cb, pyr=pyr, a0r=a0r, a1r=a1r):
            slab = xp_ref[0, r, gs:gs + Cg, :]
            return acc + contrib(r.astype(jnp.float32), slab, 0, 0, x0cb,
                                 pyr, a0r, a1r)

          acc = lax.fori_loop(base + _WIN, hi + 1, body,
                              jnp.zeros((Cg, W), jnp.float32))
          val_ref[i * Cg:(i + 1) * Cg, cs] = (
              val_ref[i * Cg:(i + 1) * Cg, cs] + acc)

  res = jnp.dot(wm_ref[...], val_ref[...],
                preferred_element_type=jnp.float32)
  for hb in range(_HB):
    out_ref[0, :, hb, :] = res[:, hb * W:(hb + 1) * W]


@jax.jit
def kernel(x, w_main, w_off, b_off, w_mask, b_mask):
  B, C, H, W = x.shape
  O = w_main.shape[0]
  Cg = C // _OG
  n_cat = 3 * _NT
  n_pad = 56

  xb = x.astype(jnp.bfloat16)
  xlo = lax.bitcast_convert_type(xb, jnp.uint16).astype(jnp.uint32)
  xnb = jnp.pad(xb[:, :, :, 1:], ((0, 0), (0, 0), (0, 0), (0, 1)))
  xhi = lax.bitcast_convert_type(xnb, jnp.uint16).astype(jnp.uint32)
  xpk = lax.bitcast_convert_type((xhi << 16) | xlo, jnp.int32)
  xp = jnp.transpose(xpk, (0, 2, 1, 3))

  w_off_r = w_off.reshape(_NT, 2, C, _K, _K)
  b_off_r = b_off.reshape(_NT, 2)
  wcat = jnp.concatenate([w_off_r[:, 0], w_off_r[:, 1], w_mask], axis=0)
  wcat = wcat.transpose(0, 2, 3, 1).reshape(n_cat, _K2 * C)
  wcat = jnp.pad(wcat, ((0, n_pad - n_cat), (0, 0)))
  bcat = jnp.concatenate([b_off_r[:, 0], b_off_r[:, 1], b_mask], axis=0)
  bcat = jnp.pad(bcat, (0, n_pad - n_cat))
  bcat = jnp.broadcast_to(bcat[:, None], (n_pad, _HB * W))

  wm = w_main.reshape(O, _OG, Cg, _K, _K)
  wm = wm.transpose(0, 1, 3, 4, 2).reshape(O, _NT * Cg)

  body = functools.partial(_dc_kernel, H=H, W=W, C=C, Cg=Cg)
  out_t = pl.pallas_call(
      body,
      grid=(B, H // _HB),
      in_specs=[
          pl.BlockSpec((1, H, C, W), lambda b, j: (b, 0, 0, 0)),
          pl.BlockSpec((n_pad, _K2 * C), lambda b, j: (0, 0)),
          pl.BlockSpec((n_pad, _HB * W), lambda b, j: (0, 0)),
          pl.BlockSpec((O, _NT * Cg), lambda b, j: (0, 0)),
      ],
      out_specs=pl.BlockSpec((1, O, _HB, W), lambda b, j: (b, 0, j, 0)),
      out_shape=jax.ShapeDtypeStruct((B, O, H, W), jnp.float32),
      scratch_shapes=[
          pltpu.VMEM((_K2 * C, _HB * W), jnp.float32),
          pltpu.VMEM((_NT * Cg, _HB * W), jnp.float32),
          pltpu.VMEM((_NT, _HB * W), jnp.int32),
          pltpu.VMEM((3 * _NT, _HB * W), jnp.float32),
      ],
      compiler_params=pltpu.CompilerParams(
          dimension_semantics=(pltpu.GridDimensionSemantics.PARALLEL,
                               pltpu.GridDimensionSemantics.ARBITRARY),
          vmem_limit_bytes=64 * 1024 * 1024,
          flags={"XLA_TPU_STORE_TO_LOAD_FORWARDING_WINDOW": 12288},
      ),
  )(xp, wcat, bcat, wm)

  return out_t


# speedup vs baseline: 1.2576x; 1.0361x over previous
import functools

import jax
import jax.numpy as jnp
from jax import lax
from jax.experimental import pallas as pl
from jax.experimental.pallas import tpu as pltpu

_K = 3
_PAD = 1
_OG = 2
_K2 = _K * _K
_NT = _OG * _K2
_WIN = 4
_HB = 16


def _dc_kernel(xp_ref, wcat_ref, bcat_ref, wm_ref, out_ref,
               patch_ref, val_ref, idx_ref, wts_ref, *, H, W, C, Cg):
  h0 = pl.program_id(1) * _HB
  WB = _HB * W

  lane_c = lax.broadcasted_iota(jnp.int32, (C, W), 1)

  shifted = {}
  for dr in range(-_PAD, _HB + _PAD):
    row = h0 + dr
    rowc = jnp.clip(row, 0, H - 1)
    slab_i = xp_ref[0, rowc, :, :]
    valid = jnp.logical_and(row >= 0, row < H)
    slab_i = jnp.where(valid, slab_i, 0)
    slab = pltpu.bitcast(slab_i << 16, jnp.float32)
    sr = pltpu.bitcast(slab_i & jnp.int32(-65536), jnp.float32)
    sl = pltpu.roll(slab, 1, axis=1)
    sl = jnp.where(lane_c < 1, 0.0, sl)
    shifted[dr] = (sl, slab, sr)
  for ki in range(_K):
    for kj in range(_K):
      r0 = (ki * _K + kj) * C
      for hb in range(_HB):
        patch_ref[r0:r0 + C, hb * W:(hb + 1) * W] = shifted[hb + ki - _PAD][kj]

  om = jnp.dot(wcat_ref[...], patch_ref[...],
               preferred_element_type=jnp.float32) + bcat_ref[...]

  dy_all = om[0:_NT, :]
  dx_all = om[_NT:2 * _NT, :]
  m_all = jax.nn.sigmoid(om[2 * _NT:3 * _NT, :])

  si = lax.broadcasted_iota(jnp.int32, (_NT, WB), 0)
  lane_b = lax.broadcasted_iota(jnp.int32, (_NT, WB), 1)
  kiv = ((si % _K2) // _K).astype(jnp.float32)
  kjv = (si % _K).astype(jnp.float32)
  hbv = (lane_b // W).astype(jnp.float32)
  wv = (lane_b % W).astype(jnp.float32)

  h0f = h0.astype(jnp.float32)
  py = dy_all + (h0f - _PAD) + hbv + kiv
  px = dx_all + (wv - _PAD) + kjv
  y0f = jnp.floor(py)
  x0f = jnp.floor(px)
  wx = px - x0f
  x0 = x0f.astype(jnp.int32)
  x1 = x0 + 1
  x0c = jnp.clip(x0, 0, W - 1)
  x1c = jnp.clip(x1, 0, W - 1)
  vx0 = jnp.where(jnp.logical_and(x0 >= 0, x0 <= W - 1), 1.0, 0.0)
  vx1 = jnp.where(jnp.logical_and(x1 >= 0, x1 <= W - 1), 1.0, 0.0)
  mwxl = (1.0 - wx) * vx0 * m_all
  mwxr = wx * vx1 * m_all
  a0 = mwxl + jnp.where(x0 == -1, mwxr, 0.0)
  a1 = jnp.where(x0 == -1, 0.0, mwxr)

  idx_ref[...] = x0c
  wts_ref[0:_NT, :] = a0
  wts_ref[_NT:2 * _NT, :] = a1
  wts_ref[2 * _NT:3 * _NT, :] = py

  base_f = []
  hi_f = []
  span = jnp.float32(0.0)
  for hb in range(_HB):
    ys = y0f[:, hb * W:(hb + 1) * W]
    ymin = jnp.min(ys, axis=1, keepdims=True)
    ymax = jnp.max(ys, axis=1, keepdims=True)
    b_ = jnp.clip(ymin, 0.0, float(H - _WIN))
    l_ = jnp.clip(ymin, 0.0, float(H - 1))
    hi_ = jnp.clip(ymax + 1.0, 0.0, float(H - 1))
    base_f.append(b_)
    hi_f.append(hi_)
    span = jnp.maximum(span, jnp.max(hi_ - l_))

  def contrib(rf, slab, i, hb, x0cb, pyr, a0r, a1r):
    tp = jnp.take_along_axis(slab, x0cb, axis=1)
    t0 = pltpu.bitcast(tp << 16, jnp.float32)
    t1 = pltpu.bitcast(tp & jnp.int32(-65536), jnp.float32)
    cy = jnp.maximum(1.0 - jnp.abs(pyr - rf), 0.0)
    la = jnp.broadcast_to(a0r * cy, (Cg, W))
    ra = jnp.broadcast_to(a1r * cy, (Cg, W))
    return t0 * la + t1 * ra

  for g in range(_OG):
    gs = g * Cg
    for k in range(_K2):
      i = g * _K2 + k
      for hb in range(_HB):
        cs = slice(hb * W, (hb + 1) * W)
        x0cb = jnp.broadcast_to(idx_ref[i:i + 1, cs], (Cg, W))
        pyr = wts_ref[2 * _NT + i:2 * _NT + i + 1, cs]
        a0r = wts_ref[i:i + 1, cs]
        a1r = wts_ref[_NT + i:_NT + i + 1, cs]
        base = base_f[hb][i, 0].astype(jnp.int32)
        slab4 = xp_ref[0, pl.ds(base, _WIN), gs:gs + Cg, :]

        acc = jnp.zeros((Cg, W), jnp.float32)
        for u in range(_WIN):
          rf = (base + u).astype(jnp.float32)
          acc = acc + contrib(rf, slab4[u], i, hb, x0cb, pyr, a0r, a1r)

        val_ref[i * Cg:(i + 1) * Cg, cs] = acc

  @pl.when(span > float(_WIN) - 0.5)
  def _residual():
    for g in range(_OG):
      gs = g * Cg
      for k in range(_K2):
        i = g * _K2 + k
        for hb in range(_HB):
          cs = slice(hb * W, (hb + 1) * W)
          x0cb = jnp.broadcast_to(idx_ref[i:i + 1, cs], (Cg, W))
          pyr = wts_ref[2 * _NT + i:2 * _NT + i + 1, cs]
          a0r = wts_ref[i:i + 1, cs]
          a1r = wts_ref[_NT + i:_NT + i + 1, cs]
          base = base_f[hb][i, 0].astype(jnp.int32)
          hi = hi_f[hb][i, 0].astype(jnp.int32)

          def body(r, acc, *, gs=gs, x0cb=x0cb, pyr=pyr, a0r=a0r, a1r=a1r):
            slab = xp_ref[0, r, gs:gs + Cg, :]
            return acc + contrib(r.astype(jnp.float32), slab, 0, 0, x0cb,
                                 pyr, a0r, a1r)

          acc = lax.fori_loop(base + _WIN, hi + 1, body,
                              jnp.zeros((Cg, W), jnp.float32))
          val_ref[i * Cg:(i + 1) * Cg, cs] = (
              val_ref[i * Cg:(i + 1) * Cg, cs] + acc)

  res = jnp.dot(wm_ref[...], val_ref[...],
                preferred_element_type=jnp.float32)
  for hb in range(_HB):
    out_ref[0, :, hb, :] = res[:, hb * W:(hb + 1) * W]


@jax.jit
def kernel(x, w_main, w_off, b_off, w_mask, b_mask):
  B, C, H, W = x.shape
  O = w_main.shape[0]
  Cg = C // _OG
  n_cat = 3 * _NT
  n_pad = 56

  xb = x.astype(jnp.bfloat16)
  xlo = lax.bitcast_convert_type(xb, jnp.uint16).astype(jnp.uint32)
  xnb = jnp.pad(xb[:, :, :, 1:], ((0, 0), (0, 0), (0, 0), (0, 1)))
  xhi = lax.bitcast_convert_type(xnb, jnp.uint16).astype(jnp.uint32)
  xpk = lax.bitcast_convert_type((xhi << 16) | xlo, jnp.int32)
  xp = jnp.transpose(xpk, (0, 2, 1, 3))

  w_off_r = w_off.reshape(_NT, 2, C, _K, _K)
  b_off_r = b_off.reshape(_NT, 2)
  wcat = jnp.concatenate([w_off_r[:, 0], w_off_r[:, 1], w_mask], axis=0)
  wcat = wcat.transpose(0, 2, 3, 1).reshape(n_cat, _K2 * C)
  wcat = jnp.pad(wcat, ((0, n_pad - n_cat), (0, 0)))
  bcat = jnp.concatenate([b_off_r[:, 0], b_off_r[:, 1], b_mask], axis=0)
  bcat = jnp.pad(bcat, (0, n_pad - n_cat))
  bcat = jnp.broadcast_to(bcat[:, None], (n_pad, _HB * W))

  wm = w_main.reshape(O, _OG, Cg, _K, _K)
  wm = wm.transpose(0, 1, 3, 4, 2).reshape(O, _NT * Cg)

  body = functools.partial(_dc_kernel, H=H, W=W, C=C, Cg=Cg)
  out_t = pl.pallas_call(
      body,
      grid=(B, H // _HB),
      in_specs=[
          pl.BlockSpec((1, H, C, W), lambda b, j: (b, 0, 0, 0)),
          pl.BlockSpec((n_pad, _K2 * C), lambda b, j: (0, 0)),
          pl.BlockSpec((n_pad, _HB * W), lambda b, j: (0, 0)),
          pl.BlockSpec((O, _NT * Cg), lambda b, j: (0, 0)),
      ],
      out_specs=pl.BlockSpec((1, O, _HB, W), lambda b, j: (b, 0, j, 0)),
      out_shape=jax.ShapeDtypeStruct((B, O, H, W), jnp.float32),
      scratch_shapes=[
          pltpu.VMEM((_K2 * C, _HB * W), jnp.float32),
          pltpu.VMEM((_NT * Cg, _HB * W), jnp.float32),
          pltpu.VMEM((_NT, _HB * W), jnp.int32),
          pltpu.VMEM((3 * _NT, _HB * W), jnp.float32),
      ],
      compiler_params=pltpu.CompilerParams(
          dimension_semantics=(pltpu.GridDimensionSemantics.PARALLEL,
                               pltpu.GridDimensionSemantics.ARBITRARY),
          vmem_limit_bytes=64 * 1024 * 1024,
      ),
  )(xp, wcat, bcat, wm)

  return out_t
